# baseline (speedup 1.0000x reference)
import functools

import numpy as np
import jax
import jax.numpy as jnp
from jax import lax
from jax.experimental import pallas as pl
from jax.experimental.pallas import tpu as pltpu
from jax.experimental.pallas import tpu_sc as plsc

_DIM = 64
_MAX_LEN = 256

NC = 2
NS = 16
NW = NC * NS

CB = 4
IDX_W = 100


def _sinusoidal_pe(max_len, dim):
    pos = np.arange(max_len, dtype=np.float32)[:, None]
    i = np.arange(0, dim, 2, dtype=np.float32)[None, :]
    angle = pos / np.power(10000.0, i / dim)
    pe = np.zeros((max_len, dim), dtype=np.float32)
    pe[:, 0::2] = np.sin(angle)
    pe[:, 1::2] = np.cos(angle)
    return pe


@functools.partial(jax.jit, static_argnums=(3, 4))
def _run(ids2, pe_rep, table, B, S):
    D = table.shape[1]
    C = CB * S
    rows_pw = B // NW
    nchunk = rows_pw // CB
    kpc = C // IDX_W

    mesh = plsc.VectorSubcoreMesh(core_axis_name="c", subcore_axis_name="s")

    @functools.partial(
        pl.kernel,
        mesh=mesh,
        out_type=jax.ShapeDtypeStruct((B, S, D), jnp.float32),
        compiler_params=pltpu.CompilerParams(use_tc_tiling_on_sc=False),
        scratch_types=[
            pltpu.VMEM((2, kpc, IDX_W), jnp.int32),
            pltpu.VMEM_SHARED((CB, S, D), jnp.float32),
            pltpu.VMEM((2, CB, S, D), jnp.float32),
            pltpu.SemaphoreType.DMA,
            pltpu.SemaphoreType.DMA,
            pltpu.SemaphoreType.DMA,
        ],
    )
    def body(ids_hbm, pe_hbm, table_hbm, out_hbm,
             idx_v, pe_sh, rows_v, isem, gsem, ssem):
        sid = lax.axis_index("s")
        wid = sid * NC + lax.axis_index("c")

        @pl.when(sid == 0)
        def _():
            pltpu.sync_copy(pe_hbm, pe_sh)

        plsc.subcore_barrier()

        idx_row_base = wid * nchunk * kpc
        row_base = wid * rows_pw
        spc = S // IDX_W

        def idx_src(c):
            return ids_hbm.at[pl.ds(idx_row_base + c * kpc, kpc)]

        def out_dst(c):
            return out_hbm.at[pl.ds(row_base + c * CB, CB)]

        def fire_gathers(c, b):
            for k in range(kpc):
                pltpu.async_copy(
                    table_hbm.at[idx_v.at[b, k]],
                    rows_v.at[b, k // spc].at[pl.ds((k % spc) * IDX_W, IDX_W)],
                    gsem, add=True)

        def wait_gathers(b):
            for k in range(kpc):
                pltpu.make_async_copy(
                    table_hbm.at[idx_v.at[b, k]],
                    rows_v.at[b, k // spc].at[pl.ds((k % spc) * IDX_W, IDX_W)],
                    gsem).wait()

        pltpu.sync_copy(idx_src(0), idx_v.at[0])
        pltpu.async_copy(idx_src(1), idx_v.at[1], isem)
        pltpu.sync_copy(pe_sh, rows_v.at[0])
        fire_gathers(0, 0)

        def half(g, b):
            c = 2 * g + b
            nb = b ^ 1

            @pl.when(c >= 1)
            def _():
                pltpu.make_async_copy(rows_v.at[nb], out_dst(c - 1),
                                      ssem).wait()

            @pl.when(c + 1 < nchunk)
            def _():
                pltpu.sync_copy(pe_sh, rows_v.at[nb])
                pltpu.make_async_copy(idx_src(c + 1), idx_v.at[nb],
                                      isem).wait()

            wait_gathers(b)
            pltpu.async_copy(rows_v.at[b], out_dst(c), ssem)

            @pl.when(c + 2 < nchunk)
            def _():
                pltpu.async_copy(idx_src(c + 2), idx_v.at[b], isem)

            @pl.when(c + 1 < nchunk)
            def _():
                fire_gathers(c + 1, nb)

        def outer(g, carry):
            half(g, 0)
            half(g, 1)
            return carry

        lax.fori_loop(0, nchunk // 2, outer, 0)
        pltpu.make_async_copy(rows_v.at[(nchunk - 1) % 2],
                              out_dst(nchunk - 1), ssem).wait()

    return body(ids2, pe_rep, table)


def kernel(input, tok_table):
    B, S = input.shape
    V, D = tok_table.shape
    pe = _sinusoidal_pe(_MAX_LEN, D)[:S]
    pe_rep = jnp.asarray(np.broadcast_to(pe, (CB, S, D)))
    ids2 = input.reshape(B * S // IDX_W, IDX_W).astype(jnp.int32)
    return _run(ids2, pe_rep, tok_table, B, S)


# speedup vs baseline: 1.1663x; 1.1663x over previous
import functools

import numpy as np
import jax
import jax.numpy as jnp
from jax import lax
from jax.experimental import pallas as pl
from jax.experimental.pallas import tpu as pltpu
from jax.experimental.pallas import tpu_sc as plsc

_DIM = 64
_MAX_LEN = 256

NC = 2
NS = 16
NW = NC * NS
BLK = 128
ITILE = 8


def _sinusoidal_pe(max_len, dim):
    pos = np.arange(max_len, dtype=np.float32)[:, None]
    i = np.arange(0, dim, 2, dtype=np.float32)[None, :]
    angle = pos / np.power(10000.0, i / dim)
    pe = np.zeros((max_len, dim), dtype=np.float32)
    pe[:, 0::2] = np.sin(angle)
    pe[:, 1::2] = np.cos(angle)
    return pe


@functools.partial(jax.jit, static_argnums=(3, 4))
def _run(ids_t, pe, table_pad, B, S):
    D = _DIM
    P = 2 * D
    ntile = S // ITILE

    mesh = plsc.VectorSubcoreMesh(core_axis_name="c", subcore_axis_name="s")

    @functools.partial(
        pl.kernel,
        mesh=mesh,
        out_type=jax.ShapeDtypeStruct((B, S, P), jnp.float32),
        scratch_types=[
            pltpu.VMEM((2, ITILE, BLK), jnp.int32),
            pltpu.VMEM((2, BLK, P), jnp.float32),
            pltpu.VMEM((S, D), jnp.float32),
            pltpu.SemaphoreType.DMA,
            pltpu.SemaphoreType.DMA,
            pltpu.SemaphoreType.DMA,
        ],
    )
    def body(ids_hbm, pe_hbm, table_hbm, out_hbm,
             idx_v, rows_v, pe_v, isem, gsem, ssem):
        wid = lax.axis_index("s") * NC + lax.axis_index("c")
        b0 = wid * BLK

        pltpu.sync_copy(pe_hbm, pe_v)

        def idx_src(t):
            return ids_hbm.at[pl.ds(t * ITILE, ITILE), pl.ds(b0, BLK)]

        def gather_refs(c, b):
            return (table_hbm.at[idx_v.at[(c // ITILE) % 2, c % ITILE]],
                    rows_v.at[b])

        def store_refs(c, b):
            return (rows_v.at[b], out_hbm.at[pl.ds(b0, BLK), c, :])

        def gather(c, b):
            pltpu.async_copy(*gather_refs(c, b), gsem)

        def wait_gather(c, b):
            pltpu.make_async_copy(*gather_refs(c, b), gsem).wait()

        def store(c, b):
            pltpu.async_copy(*store_refs(c, b), ssem)

        def wait_store(c, b):
            pltpu.make_async_copy(*store_refs(c, b), ssem).wait()

        pltpu.sync_copy(idx_src(0), idx_v.at[0])
        pltpu.async_copy(idx_src(1), idx_v.at[1], isem)
        gather(0, 0)

        def half(g, b):
            c = 2 * g + b
            nb = b ^ 1

            @pl.when(c >= 1)
            def _():
                wait_store(c - 1, nb)

            @pl.when(jnp.logical_and(c + 1 < S, (c + 1) % ITILE == 0))
            def _():
                pltpu.make_async_copy(idx_src((c + 1) // ITILE),
                                      idx_v.at[((c + 1) // ITILE) % 2],
                                      isem).wait()

            wait_gather(c, b)

            @pl.when(c + 1 < S)
            def _():
                gather(c + 1, nb)

            @pl.when(jnp.logical_and((c + 1) % ITILE == 0,
                                     (c + 1) // ITILE + 1 < ntile))
            def _():
                pltpu.async_copy(idx_src((c + 1) // ITILE + 1),
                                 idx_v.at[(c // ITILE) % 2], isem)

            pr = [pe_v[c, pl.ds(16 * j, 16)] for j in range(D // 16)]

            def add_row(r, carry):
                for u in range(2):
                    for j in range(D // 16):
                        row = 2 * r + u
                        col = pl.ds(16 * j, 16)
                        rows_v[b, row, col] = rows_v[b, row, col] + pr[j]
                return carry

            lax.fori_loop(0, BLK // 2, add_row, 0)
            store(c, b)

        def outer(g, carry):
            half(g, 0)
            half(g, 1)
            return carry

        lax.fori_loop(0, S // 2, outer, 0)
        wait_store(S - 1, 1)

    return body(ids_t, pe, table_pad)


def kernel(input, tok_table):
    B, S = input.shape
    V, D = tok_table.shape
    pe = jnp.asarray(_sinusoidal_pe(_MAX_LEN, D)[:S])
    ids_t = input.T.astype(jnp.int32)
    table_pad = jnp.pad(tok_table, ((0, 0), (0, D)))
    return _run(ids_t, pe, table_pad, B, S)[:, :, :D]
